# baseline (speedup 1.0000x reference)
import functools

import jax
import jax.numpy as jnp
from jax import lax
from jax.experimental import pallas as pl
from jax.experimental.pallas import tpu as pltpu
from jax.experimental.pallas import tpu_sc as plsc

_info = plsc.get_sparse_core_info()
_NC, _NS = _info.num_cores, _info.num_subcores
_NW = _NC * _NS

_CHUNK = 128
_NBUF = 3
_LEAD = 2


def _make_gather(B: int, D: int):
    b_per_w = B // _NW
    n_chunks = b_per_w // _CHUNK
    n_main = (n_chunks // _NBUF) * _NBUF

    mesh = plsc.VectorSubcoreMesh(core_axis_name="c", subcore_axis_name="s")

    @functools.partial(
        pl.kernel,
        out_type=jax.ShapeDtypeStruct((B, D), jnp.float32),
        mesh=mesh,
        scratch_types=[
            pltpu.VMEM((n_chunks, _CHUNK), jnp.int32),
            [pltpu.VMEM((_CHUNK, D), jnp.float32) for _ in range(_NBUF)],
            [pltpu.VMEM_SHARED((_NS * _CHUNK, D), jnp.float32) for _ in range(_NBUF)],
            [pltpu.SemaphoreType.DMA for _ in range(_NBUF)],
            [pltpu.SemaphoreType.DMA for _ in range(_NBUF)],
            [pltpu.SemaphoreType.DMA for _ in range(_NBUF)],
        ],
    )
    def gather_kernel(
        table_hbm, idx_hbm, out_hbm, idx_v, rows, shared, g_sems, x_sems, o_sems
    ):
        s = lax.axis_index("s")
        wid = s * _NC + lax.axis_index("c")
        out_base = wid * b_per_w

        pltpu.sync_copy(idx_hbm.at[pl.ds(wid * n_chunks, n_chunks)], idx_v)

        def sh(b):
            return shared[b].at[pl.ds(s * _CHUNK, _CHUNK)]

        def start_gather(j, b):
            pltpu.async_copy(table_hbm.at[idx_v.at[j]], rows[b], g_sems[b])

        def wait_gather(j, b):
            pltpu.make_async_copy(table_hbm.at[idx_v.at[j]], rows[b], g_sems[b]).wait()

        def start_xbar(j, b):
            pltpu.async_copy(rows[b], sh(b), x_sems[b])

        def wait_xbar(b):
            pltpu.make_async_copy(rows[b], sh(b), x_sems[b]).wait()

        def start_out(j, b):
            pltpu.async_copy(
                sh(b), out_hbm.at[pl.ds(out_base + j * _CHUNK, _CHUNK)], o_sems[b]
            )

        def wait_out(b):
            pltpu.make_async_copy(
                sh(b), out_hbm.at[pl.ds(out_base, _CHUNK)], o_sems[b]
            ).wait()

        for j in range(_LEAD):
            start_gather(j, j)

        def step(j, b, first_group, last_group):
            wait_gather(j, b)
            if not first_group:

                @pl.when(j >= _NBUF)
                def _():
                    wait_out(b)

            start_xbar(j, b)
            bp = (b - 1) % _NBUF
            if first_group:
                if j >= 1:
                    wait_xbar(bp)
                    start_out(j - 1, bp)
            else:
                wait_xbar(bp)
                start_out(j - 1, bp)
            jn = j + _LEAD
            bn = (b + _LEAD) % _NBUF
            if last_group:
                if jn < n_chunks:
                    start_gather(jn, bn)
            else:

                @pl.when(jn < n_chunks)
                def _():
                    start_gather(jn, bn)

        def body(g, carry):
            for b in range(_NBUF):
                j = g * _NBUF + b
                step(j, b, False, False)
            return carry

        for b in range(_NBUF):
            step(b, b, True, False)

        lax.fori_loop(1, n_main // _NBUF, body, 0)

        for j in range(n_main, n_chunks):
            step(j, j % _NBUF, False, True)

        bl = (n_chunks - 1) % _NBUF
        wait_xbar(bl)
        start_out(n_chunks - 1, bl)
        for b in range(_NBUF):
            wait_out(b)

    return gather_kernel


def kernel(item_ids, table):
    ids_shape = item_ids.shape
    B = ids_shape[0] * ids_shape[1]
    D = table.shape[1]
    idx2d = item_ids.reshape(B // _CHUNK, _CHUNK).astype(jnp.int32)
    out = _make_gather(B, D)(table, idx2d)
    return out.reshape(*ids_shape, D)
